# baseline (speedup 1.0000x reference)
import functools

import jax
import jax.numpy as jnp
from jax import lax
from jax.experimental import pallas as pl
from jax.experimental.pallas import tpu as pltpu
from jax.experimental.pallas import tpu_sc as plsc

D_MODEL = 1024
N_FEATURES = 16384
BATCH = 4096
TOPK = 64
FULL_BATCH_SIZE = 4096
GRAD_CLIP_THRESHOLD = 1.0

_NS = 16
_IDX_TOTAL = BATCH * TOPK
_IDX_ROWS = _IDX_TOTAL // 128
_ROWS_PER_W = _IDX_ROWS // _NS
_BINS_PER_W = N_FEATURES // _NS
_BATCH_DMAS = 16


def _sc_histogram(idx2d, zeros_init, blend, fd, ai):
    mesh = plsc.VectorSubcoreMesh(core_axis_name="c", subcore_axis_name="s",
                                  num_cores=1)

    @functools.partial(
        pl.kernel,
        out_type=[jax.ShapeDtypeStruct((N_FEATURES,), jnp.float32),
                  jax.ShapeDtypeStruct((N_FEATURES,), jnp.int32)],
        mesh=mesh,
        scratch_types=[
            pltpu.VMEM((_ROWS_PER_W, 128), jnp.int32),
            pltpu.VMEM((128,), jnp.float32),
            pltpu.VMEM((32,), jnp.float32),
            pltpu.VMEM((_BINS_PER_W,), jnp.float32),
            pltpu.VMEM((_BINS_PER_W,), jnp.float32),
            pltpu.VMEM((_BINS_PER_W,), jnp.int32),
            pltpu.VMEM_SHARED((N_FEATURES,), jnp.float32),
            pltpu.SemaphoreType.DMA,
        ],
    )
    def hist(idx_hbm, zero_hbm, blend_hbm, fd_hbm, ai_hbm,
             outfd_hbm, outai_hbm,
             idx_v, ones_v, blend_v, cnt_v, fd_v, ai_v, counts_sh, sem):
        sid = lax.axis_index("s")

        pltpu.sync_copy(idx_hbm.at[pl.ds(sid * _ROWS_PER_W, _ROWS_PER_W)],
                        idx_v)
        pltpu.sync_copy(blend_hbm, blend_v)

        for i in range(8):
            ones_v[pl.ds(i * 16, 16)] = jnp.ones((16,), jnp.float32)

        @pl.when(sid == 0)
        def _():
            pltpu.sync_copy(zero_hbm, counts_sh)

        plsc.subcore_barrier()

        for b in range(_ROWS_PER_W // _BATCH_DMAS):
            descs = [
                pltpu.async_copy(
                    ones_v, counts_sh.at[idx_v.at[b * _BATCH_DMAS + j]], sem,
                    add=True)
                for j in range(_BATCH_DMAS)
            ]
            for d in descs:
                d.wait()

        plsc.subcore_barrier()

        base = sid * _BINS_PER_W
        pltpu.sync_copy(counts_sh.at[pl.ds(base, _BINS_PER_W)], cnt_v)
        pltpu.sync_copy(fd_hbm.at[pl.ds(base, _BINS_PER_W)], fd_v)
        pltpu.sync_copy(ai_hbm.at[pl.ds(base, _BINS_PER_W)], ai_v)

        wfv = blend_v[pl.ds(0, 16)]
        nwfv = blend_v[pl.ds(16, 16)]
        inv_fb = jnp.float32(1.0 / FULL_BATCH_SIZE)

        def bin_body(k, carry):
            sl = pl.ds(k * 16, 16)
            c = cnt_v[sl]
            fd_v[sl] = fd_v[sl] * wfv + (c * inv_fb) * nwfv
            ai_v[sl] = jnp.where(c > 0.0, 0, ai_v[sl] + 1)
            return carry

        lax.fori_loop(0, _BINS_PER_W // 16, bin_body, 0)

        pltpu.sync_copy(fd_v, outfd_hbm.at[pl.ds(base, _BINS_PER_W)])
        pltpu.sync_copy(ai_v, outai_hbm.at[pl.ds(base, _BINS_PER_W)])

    return hist(idx2d, zeros_init, blend, fd, ai)


_G = 32
_FBLK = N_FEATURES // _G
_DBLK = D_MODEL // _G
_XBLK = BATCH // _G

_R_XN, _R_SE, _R_SD, _R_GE, _R_GD = 0, 1, 2, 3, 4


def _reduce_body(scal_ref, xb, we, wd, ge, gd, scal_out, acc):
    i = pl.program_id(0)

    @pl.when(i == 0)
    def _():
        acc[...] = jnp.zeros_like(acc)

    xs = xb[...]
    rn = jnp.sqrt(jnp.sum(xs * xs, axis=1))
    acc[_R_XN, :] += jnp.sum(rn.reshape(-1, 128), axis=0)

    def _sumsq_cols(ref):
        v = ref[...]
        p = jnp.sum(v * v, axis=0)
        return jnp.sum(p.reshape(-1, 128), axis=0)

    acc[_R_SE, :] += _sumsq_cols(we)
    acc[_R_SD, :] += _sumsq_cols(wd)
    acc[_R_GE, :] += _sumsq_cols(ge)
    acc[_R_GD, :] += _sumsq_cols(gd)

    @pl.when(i == _G - 1)
    def _():
        wf = scal_ref[0]
        nwf = scal_ref[1]
        old_avg = scal_ref[2]
        old_gcp = scal_ref[3]

        tot = jnp.sum(acc[...], axis=1, keepdims=True)
        sx = tot[_R_XN:_R_XN + 1, :]
        se = tot[_R_SE:_R_SE + 1, :]
        sd = tot[_R_SD:_R_SD + 1, :]
        geh = tot[_R_GE:_R_GE + 1, :]
        gdh = tot[_R_GD:_R_GD + 1, :]

        avg_new = old_avg * wf + (sx / BATCH) * nwf
        gnorm = jnp.sqrt(geh + gdh)
        gcp_new = (old_gcp * wf +
                   (gnorm > GRAD_CLIP_THRESHOLD).astype(jnp.float32) * nwf)

        scal_out[0:1, :] = jnp.broadcast_to(avg_new, (1, 128))
        scal_out[1:2, :] = jnp.broadcast_to(gcp_new, (1, 128))
        scal_out[2:3, :] = jnp.broadcast_to(jnp.sqrt(se), (1, 128))
        scal_out[3:4, :] = jnp.broadcast_to(jnp.sqrt(sd), (1, 128))
        scal_out[4:5, :] = jnp.broadcast_to(jnp.sqrt(geh), (1, 128))
        scal_out[5:6, :] = jnp.broadcast_to(jnp.sqrt(gdh), (1, 128))
        scal_out[6:8, :] = jnp.zeros((2, 128), jnp.float32)


def _tc_reduce(scal, x, W_enc, W_dec, gW_enc, gW_dec):
    return pl.pallas_call(
        _reduce_body,
        grid=(_G,),
        in_specs=[
            pl.BlockSpec(memory_space=pltpu.SMEM),
            pl.BlockSpec((_XBLK, D_MODEL), lambda i: (i, 0)),
            pl.BlockSpec((_DBLK, N_FEATURES), lambda i: (i, 0)),
            pl.BlockSpec((_FBLK, D_MODEL), lambda i: (i, 0)),
            pl.BlockSpec((_DBLK, N_FEATURES), lambda i: (i, 0)),
            pl.BlockSpec((_FBLK, D_MODEL), lambda i: (i, 0)),
        ],
        out_specs=pl.BlockSpec((8, 128), lambda i: (0, 0)),
        out_shape=jax.ShapeDtypeStruct((8, 128), jnp.float32),
        scratch_shapes=[pltpu.VMEM((8, 128), jnp.float32)],
        compiler_params=pltpu.CompilerParams(
            dimension_semantics=("arbitrary",),
        ),
    )(scal, x, W_enc, W_dec, gW_enc, gW_dec)


def kernel(x, k_indices, W_enc, W_dec, grad_W_enc, grad_W_dec, avg_norm,
           feature_density, activated_in, grad_clip_percent, n_steps):
    n_steps_f = jnp.asarray(n_steps, jnp.float32)
    wf = n_steps_f / (n_steps_f + 1.0)
    nwf = 1.0 / (n_steps_f + 1.0)

    scal = jnp.stack([wf, nwf,
                      jnp.asarray(avg_norm, jnp.float32),
                      jnp.asarray(grad_clip_percent, jnp.float32)])
    sc_out = _tc_reduce(scal, x, W_enc, W_dec, grad_W_enc, grad_W_dec)

    idx2d = k_indices.reshape(_IDX_ROWS, 128)
    zeros_init = jnp.zeros((N_FEATURES,), jnp.float32)
    blend = jnp.concatenate([jnp.full((16,), wf, jnp.float32),
                             jnp.full((16,), nwf, jnp.float32)])
    fd_out, ai_out = _sc_histogram(idx2d, zeros_init, blend,
                                   feature_density, activated_in)

    new_n_steps = jnp.asarray(n_steps + 1, dtype=jnp.int32)
    return (new_n_steps,
            sc_out[0, 0],
            fd_out,
            ai_out,
            sc_out[1, 0],
            sc_out[2, 0],
            sc_out[3, 0],
            sc_out[4, 0],
            sc_out[5, 0])


# speedup vs baseline: 1.0057x; 1.0057x over previous
import functools

import jax
import jax.numpy as jnp
from jax import lax
from jax.experimental import pallas as pl
from jax.experimental.pallas import tpu as pltpu
from jax.experimental.pallas import tpu_sc as plsc

D_MODEL = 1024
N_FEATURES = 16384
BATCH = 4096
TOPK = 64
FULL_BATCH_SIZE = 4096
GRAD_CLIP_THRESHOLD = 1.0

_NS = 16
_IDX_TOTAL = BATCH * TOPK
_IDX_ROWS = _IDX_TOTAL // 128
_ROWS_PER_W = _IDX_ROWS // _NS
_BINS_PER_W = N_FEATURES // _NS
_BATCH_DMAS = 16


def _sc_histogram(idx2d, zeros_init, blend, fd, ai):
    mesh = plsc.VectorSubcoreMesh(core_axis_name="c", subcore_axis_name="s",
                                  num_cores=1)

    @functools.partial(
        pl.kernel,
        out_type=[jax.ShapeDtypeStruct((N_FEATURES,), jnp.float32),
                  jax.ShapeDtypeStruct((N_FEATURES,), jnp.int32)],
        mesh=mesh,
        scratch_types=[
            pltpu.VMEM((_ROWS_PER_W, 128), jnp.int32),
            pltpu.VMEM((128,), jnp.float32),
            pltpu.VMEM((32,), jnp.float32),
            pltpu.VMEM((_BINS_PER_W,), jnp.float32),
            pltpu.VMEM((_BINS_PER_W,), jnp.float32),
            pltpu.VMEM((_BINS_PER_W,), jnp.int32),
            pltpu.VMEM_SHARED((N_FEATURES,), jnp.float32),
            pltpu.SemaphoreType.DMA,
        ],
    )
    def hist(idx_hbm, zero_hbm, blend_hbm, fd_hbm, ai_hbm,
             outfd_hbm, outai_hbm,
             idx_v, ones_v, blend_v, cnt_v, fd_v, ai_v, counts_sh, sem):
        sid = lax.axis_index("s")

        pltpu.sync_copy(idx_hbm.at[pl.ds(sid * _ROWS_PER_W, _ROWS_PER_W)],
                        idx_v)
        pltpu.sync_copy(blend_hbm, blend_v)

        for i in range(8):
            ones_v[pl.ds(i * 16, 16)] = jnp.ones((16,), jnp.float32)

        @pl.when(sid == 0)
        def _():
            pltpu.sync_copy(zero_hbm, counts_sh)

        plsc.subcore_barrier()

        def scat_body(j, carry):
            pltpu.sync_copy(ones_v, counts_sh.at[idx_v.at[j]], add=True)
            return carry

        lax.fori_loop(0, _ROWS_PER_W, scat_body, 0)

        plsc.subcore_barrier()

        base = sid * _BINS_PER_W
        pltpu.sync_copy(counts_sh.at[pl.ds(base, _BINS_PER_W)], cnt_v)
        pltpu.sync_copy(fd_hbm.at[pl.ds(base, _BINS_PER_W)], fd_v)
        pltpu.sync_copy(ai_hbm.at[pl.ds(base, _BINS_PER_W)], ai_v)

        wfv = blend_v[pl.ds(0, 16)]
        nwfv = blend_v[pl.ds(16, 16)]
        inv_fb = jnp.float32(1.0 / FULL_BATCH_SIZE)

        def bin_body(k, carry):
            sl = pl.ds(k * 16, 16)
            c = cnt_v[sl]
            fd_v[sl] = fd_v[sl] * wfv + (c * inv_fb) * nwfv
            ai_v[sl] = jnp.where(c > 0.0, 0, ai_v[sl] + 1)
            return carry

        lax.fori_loop(0, _BINS_PER_W // 16, bin_body, 0)

        pltpu.sync_copy(fd_v, outfd_hbm.at[pl.ds(base, _BINS_PER_W)])
        pltpu.sync_copy(ai_v, outai_hbm.at[pl.ds(base, _BINS_PER_W)])

    return hist(idx2d, zeros_init, blend, fd, ai)


_G = 32
_FBLK = N_FEATURES // _G
_DBLK = D_MODEL // _G
_XBLK = BATCH // _G

_R_XN, _R_SE, _R_SD, _R_GE, _R_GD = 0, 1, 2, 3, 4


def _reduce_body(scal_ref, xb, we, wd, ge, gd, scal_out, acc):
    i = pl.program_id(0)

    @pl.when(i == 0)
    def _():
        acc[...] = jnp.zeros_like(acc)

    xs = xb[...]
    rn = jnp.sqrt(jnp.sum(xs * xs, axis=1))
    acc[_R_XN, :] += jnp.sum(rn.reshape(-1, 128), axis=0)

    def _sumsq_cols(ref):
        v = ref[...]
        p = jnp.sum(v * v, axis=0)
        return jnp.sum(p.reshape(-1, 128), axis=0)

    acc[_R_SE, :] += _sumsq_cols(we)
    acc[_R_SD, :] += _sumsq_cols(wd)
    acc[_R_GE, :] += _sumsq_cols(ge)
    acc[_R_GD, :] += _sumsq_cols(gd)

    @pl.when(i == _G - 1)
    def _():
        wf = scal_ref[0]
        nwf = scal_ref[1]
        old_avg = scal_ref[2]
        old_gcp = scal_ref[3]

        tot = jnp.sum(acc[...], axis=1, keepdims=True)
        sx = tot[_R_XN:_R_XN + 1, :]
        se = tot[_R_SE:_R_SE + 1, :]
        sd = tot[_R_SD:_R_SD + 1, :]
        geh = tot[_R_GE:_R_GE + 1, :]
        gdh = tot[_R_GD:_R_GD + 1, :]

        avg_new = old_avg * wf + (sx / BATCH) * nwf
        gnorm = jnp.sqrt(geh + gdh)
        gcp_new = (old_gcp * wf +
                   (gnorm > GRAD_CLIP_THRESHOLD).astype(jnp.float32) * nwf)

        scal_out[0:1, :] = jnp.broadcast_to(avg_new, (1, 128))
        scal_out[1:2, :] = jnp.broadcast_to(gcp_new, (1, 128))
        scal_out[2:3, :] = jnp.broadcast_to(jnp.sqrt(se), (1, 128))
        scal_out[3:4, :] = jnp.broadcast_to(jnp.sqrt(sd), (1, 128))
        scal_out[4:5, :] = jnp.broadcast_to(jnp.sqrt(geh), (1, 128))
        scal_out[5:6, :] = jnp.broadcast_to(jnp.sqrt(gdh), (1, 128))
        scal_out[6:8, :] = jnp.zeros((2, 128), jnp.float32)


def _tc_reduce(scal, x, W_enc, W_dec, gW_enc, gW_dec):
    return pl.pallas_call(
        _reduce_body,
        grid=(_G,),
        in_specs=[
            pl.BlockSpec(memory_space=pltpu.SMEM),
            pl.BlockSpec((_XBLK, D_MODEL), lambda i: (i, 0)),
            pl.BlockSpec((_DBLK, N_FEATURES), lambda i: (i, 0)),
            pl.BlockSpec((_FBLK, D_MODEL), lambda i: (i, 0)),
            pl.BlockSpec((_DBLK, N_FEATURES), lambda i: (i, 0)),
            pl.BlockSpec((_FBLK, D_MODEL), lambda i: (i, 0)),
        ],
        out_specs=pl.BlockSpec((8, 128), lambda i: (0, 0)),
        out_shape=jax.ShapeDtypeStruct((8, 128), jnp.float32),
        scratch_shapes=[pltpu.VMEM((8, 128), jnp.float32)],
        compiler_params=pltpu.CompilerParams(
            dimension_semantics=("arbitrary",),
        ),
    )(scal, x, W_enc, W_dec, gW_enc, gW_dec)


def kernel(x, k_indices, W_enc, W_dec, grad_W_enc, grad_W_dec, avg_norm,
           feature_density, activated_in, grad_clip_percent, n_steps):
    n_steps_f = jnp.asarray(n_steps, jnp.float32)
    wf = n_steps_f / (n_steps_f + 1.0)
    nwf = 1.0 / (n_steps_f + 1.0)

    scal = jnp.stack([wf, nwf,
                      jnp.asarray(avg_norm, jnp.float32),
                      jnp.asarray(grad_clip_percent, jnp.float32)])
    sc_out = _tc_reduce(scal, x, W_enc, W_dec, grad_W_enc, grad_W_dec)

    idx2d = k_indices.reshape(_IDX_ROWS, 128)
    zeros_init = jnp.zeros((N_FEATURES,), jnp.float32)
    blend = jnp.concatenate([jnp.full((16,), wf, jnp.float32),
                             jnp.full((16,), nwf, jnp.float32)])
    fd_out, ai_out = _sc_histogram(idx2d, zeros_init, blend,
                                   feature_density, activated_in)

    new_n_steps = jnp.asarray(n_steps + 1, dtype=jnp.int32)
    return (new_n_steps,
            sc_out[0, 0],
            fd_out,
            ai_out,
            sc_out[1, 0],
            sc_out[2, 0],
            sc_out[3, 0],
            sc_out[4, 0],
            sc_out[5, 0])
